# Initial kernel scaffold; baseline (speedup 1.0000x reference)
import jax
import jax.numpy as jnp
from jax.experimental import pallas as pl


def kernel(x, adj_matrix, W_self, b_self, W_neigh, b_neigh):
    raise NotImplementedError("write your pallas kernel here")


# speedup vs baseline: 1.1204x; 1.1204x over previous
import functools

import jax
import jax.numpy as jnp
from jax.experimental import pallas as pl

IN_F = 128
OUT_F = 128
B = 2
N = 4096
TI = 512


def _fused_kernel(adj_ref, xh_ref, ws_ref, wn_ref, bs_ref, bn_ref, out_ref):
    i = pl.program_id(0)
    af = (adj_ref[...] > 0).astype(jnp.float32)
    deg = jnp.sum(af, axis=1, keepdims=True)
    xh = xh_ref[...]
    agg = jnp.dot(af, xh, preferred_element_type=jnp.float32)
    mean = agg / jnp.maximum(deg, 1.0)
    neigh = jnp.dot(mean, wn_ref[...], preferred_element_type=jnp.float32)
    neigh = neigh + bn_ref[...]
    neigh = jnp.where(deg > 0.0, neigh, 0.0)
    xs = xh_ref[pl.ds(i * TI, TI), :]
    self_out = jnp.dot(xs, ws_ref[...], preferred_element_type=jnp.float32)
    self_out = self_out + bs_ref[...]
    out_ref[...] = jnp.maximum(self_out + neigh, 0.0)


@jax.jit
def kernel(x, adj_matrix, W_self, b_self, W_neigh, b_neigh):
    xh = x.transpose(1, 0, 2).reshape(N, B * IN_F)
    zero = jnp.zeros((OUT_F, OUT_F), jnp.float32)
    wbd_self = jnp.block([[W_self.T, zero], [zero, W_self.T]])
    wbd_neigh = jnp.block([[W_neigh.T, zero], [zero, W_neigh.T]])
    bbd_self = jnp.concatenate([b_self, b_self]).reshape(1, B * OUT_F)
    bbd_neigh = jnp.concatenate([b_neigh, b_neigh]).reshape(1, B * OUT_F)

    out = pl.pallas_call(
        _fused_kernel,
        grid=(N // TI,),
        in_specs=[
            pl.BlockSpec((TI, N), lambda i: (i, 0)),
            pl.BlockSpec((N, B * IN_F), lambda i: (0, 0)),
            pl.BlockSpec((B * IN_F, B * OUT_F), lambda i: (0, 0)),
            pl.BlockSpec((B * IN_F, B * OUT_F), lambda i: (0, 0)),
            pl.BlockSpec((1, B * OUT_F), lambda i: (0, 0)),
            pl.BlockSpec((1, B * OUT_F), lambda i: (0, 0)),
        ],
        out_specs=pl.BlockSpec((TI, B * OUT_F), lambda i: (i, 0)),
        out_shape=jax.ShapeDtypeStruct((N, B * OUT_F), jnp.float32),
    )(adj_matrix, xh, wbd_self, wbd_neigh, bbd_self, bbd_neigh)

    return out.reshape(N, B, OUT_F).transpose(1, 0, 2)


# speedup vs baseline: 1.1264x; 1.0054x over previous
import functools

import jax
import jax.numpy as jnp
from jax.experimental import pallas as pl

IN_F = 128
OUT_F = 128
B = 2
N = 4096
TI = 512


def _fused_kernel(adj_ref, xh_ref, ws_ref, wn_ref, bs_ref, bn_ref, out_ref):
    i = pl.program_id(0)
    ab = (adj_ref[...] > 0).astype(jnp.bfloat16)
    deg = jnp.sum(ab.astype(jnp.float32), axis=1, keepdims=True)
    xh = xh_ref[...]
    agg = jnp.dot(ab, xh.astype(jnp.bfloat16),
                  preferred_element_type=jnp.float32)
    mean = agg / jnp.maximum(deg, 1.0)
    neigh = jnp.dot(mean, wn_ref[...], preferred_element_type=jnp.float32)
    neigh = neigh + bn_ref[...]
    neigh = jnp.where(deg > 0.0, neigh, 0.0)
    xs = xh_ref[pl.ds(i * TI, TI), :]
    self_out = jnp.dot(xs, ws_ref[...], preferred_element_type=jnp.float32)
    self_out = self_out + bs_ref[...]
    out_ref[...] = jnp.maximum(self_out + neigh, 0.0)


@jax.jit
def kernel(x, adj_matrix, W_self, b_self, W_neigh, b_neigh):
    xh = x.transpose(1, 0, 2).reshape(N, B * IN_F)
    zero = jnp.zeros((OUT_F, OUT_F), jnp.float32)
    wbd_self = jnp.block([[W_self.T, zero], [zero, W_self.T]])
    wbd_neigh = jnp.block([[W_neigh.T, zero], [zero, W_neigh.T]])
    bbd_self = jnp.concatenate([b_self, b_self]).reshape(1, B * OUT_F)
    bbd_neigh = jnp.concatenate([b_neigh, b_neigh]).reshape(1, B * OUT_F)

    out = pl.pallas_call(
        _fused_kernel,
        grid=(N // TI,),
        in_specs=[
            pl.BlockSpec((TI, N), lambda i: (i, 0)),
            pl.BlockSpec((N, B * IN_F), lambda i: (0, 0)),
            pl.BlockSpec((B * IN_F, B * OUT_F), lambda i: (0, 0)),
            pl.BlockSpec((B * IN_F, B * OUT_F), lambda i: (0, 0)),
            pl.BlockSpec((1, B * OUT_F), lambda i: (0, 0)),
            pl.BlockSpec((1, B * OUT_F), lambda i: (0, 0)),
        ],
        out_specs=pl.BlockSpec((TI, B * OUT_F), lambda i: (i, 0)),
        out_shape=jax.ShapeDtypeStruct((N, B * OUT_F), jnp.float32),
    )(adj_matrix, xh, wbd_self, wbd_neigh, bbd_self, bbd_neigh)

    return out.reshape(N, B, OUT_F).transpose(1, 0, 2)
